# baseline (speedup 1.0000x reference)
import functools

import jax
import jax.numpy as jnp
from jax.experimental import pallas as pl
from jax.experimental.pallas import tpu as pltpu

_T_RANGE = 30.0
_N_BISECT = 20


def _sigmoid(x):
    return 0.5 + 0.5 * jnp.tanh(0.5 * x)


def _topk_mean(pre, len_b, l):
    c = pre.shape[1]
    rows = jax.lax.broadcasted_iota(jnp.int32, (l, 1), 0)
    pre_m = jnp.where(rows < len_b, pre, -jnp.inf)
    k_f = ((len_b + jnp.int32(7)) // jnp.int32(8)).astype(jnp.float32)

    lo0 = jnp.full((1, c), -_T_RANGE, jnp.float32)
    hi0 = jnp.full((1, c), _T_RANGE, jnp.float32)

    def body(_, carry):
        lo, hi = carry
        mid = 0.5 * (lo + hi)
        cnt = jnp.sum(jnp.where(pre_m > mid, 1.0, 0.0).astype(jnp.float32),
                      axis=0, keepdims=True)
        ge = cnt >= k_f
        return jnp.where(ge, mid, lo), jnp.where(ge, hi, mid)

    _, t = jax.lax.fori_loop(0, _N_BISECT, body, (lo0, hi0))

    sig = _sigmoid(pre)
    gt = pre_m > t
    cnt_gt = jnp.sum(gt.astype(jnp.float32), axis=0, keepdims=True)
    sum_gt = jnp.sum(jnp.where(gt, sig, jnp.float32(0.0)), axis=0, keepdims=True)
    t_sig = _sigmoid(t)
    return (sum_gt + (k_f - cnt_gt) * t_sig) / k_f


def _pool_chunked(pre_full, bias, len_b, l, chunk):
    c = pre_full.shape[1]
    outs = []
    for c0 in range(0, c, chunk):
        outs.append(_topk_mean(pre_full[:, c0:c0 + chunk]
                               + bias[:, c0:c0 + chunk], len_b, l))
    return jnp.concatenate(outs, axis=1) if len(outs) > 1 else outs[0]


def _mega_body(lens_ref, ba_ref, x_ref, w0_ref, b0_ref, w1_ref, b1_ref,
               w2_ref, b2_ref, wa_ref, wv0_ref, wv1_ref, wv2_ref, vb_ref,
               wn_ref, nb_ref,
               e1_ref, e2_ref, sent_ref, ilv_ref, iln_ref,
               m_s, s_s, acc_s, prev_s, edge_s, pv_s, pn_s,
               *, tl, nl, l, cchunk):
    b = pl.program_id(0)
    lt = pl.program_id(1)
    d = w0_ref.shape[1]

    @pl.when(lt == 0)
    def _init():
        m_s[0, 0] = -jnp.inf
        s_s[0, 0] = jnp.float32(0.0)
        acc_s[...] = jnp.zeros_like(acc_s)

    x = x_ref[0].astype(jnp.bfloat16)
    e0 = jnp.dot(x, w0_ref[...], preferred_element_type=jnp.float32) + b0_ref[...]
    e1 = jnp.dot(x, w1_ref[...], preferred_element_type=jnp.float32) + b1_ref[...]
    e2 = jnp.dot(x, w2_ref[...], preferred_element_type=jnp.float32) + b2_ref[...]
    e1_ref[0] = e1
    e2_ref[0] = e2
    e1b = e1.astype(jnp.bfloat16)

    a = jnp.dot(e0, wa_ref[...], preferred_element_type=jnp.float32) + ba_ref[0]
    rows = jax.lax.broadcasted_iota(jnp.int32, (tl, 1), 0) + lt * tl
    a = jnp.where(rows >= lens_ref[b], jnp.float32(-1e18), a)
    m_prev = m_s[0, 0]
    m_new = jnp.maximum(m_prev, jnp.max(a))
    alpha = jnp.exp(m_prev - m_new)
    p = jnp.exp(a - m_new)
    s_new = s_s[0, 0] * alpha + jnp.sum(p)
    acc_new = acc_s[...] * alpha + jnp.sum(p * e0, axis=0, keepdims=True)
    m_s[0, 0] = m_new
    s_s[0, 0] = s_new
    acc_s[...] = acc_new

    pn_s[pl.ds(lt * tl, tl)] = jnp.dot(
        e2.astype(jnp.bfloat16), wn_ref[...], preferred_element_type=jnp.float32)

    def conv_window(e1h):
        dm = jnp.dot(e1h, wv0_ref[...], preferred_element_type=jnp.float32)
        de = jnp.dot(e1h, wv1_ref[...], preferred_element_type=jnp.float32)
        df = jnp.dot(e1h, wv2_ref[...], preferred_element_type=jnp.float32)
        return dm[0:tl] + de[1:tl + 1] + df[2:tl + 2]

    @pl.when(lt > 0)
    def _conv_prev():
        e1h = jnp.concatenate([edge_s[...], prev_s[...], e1b[0:1]], axis=0)
        pv_s[pl.ds((lt - 1) * tl, tl)] = conv_window(e1h)
        edge_s[...] = prev_s[tl - 1:tl]

    @pl.when(lt == 0)
    def _zero_edge():
        edge_s[...] = jnp.zeros((1, d), jnp.bfloat16)

    prev_s[...] = e1b

    @pl.when(lt == nl - 1)
    def _fin():
        zrow = jnp.zeros((1, d), jnp.bfloat16)
        e1h = jnp.concatenate([edge_s[...], e1b, zrow], axis=0)
        pv_s[pl.ds((nl - 1) * tl, tl)] = conv_window(e1h)

        sent_ref[0] = acc_new / s_new
        len_b = lens_ref[b]
        ilv_ref[0] = _pool_chunked(pv_s[...], vb_ref[...], len_b, l, cchunk)
        iln_ref[0] = _pool_chunked(pn_s[...], nb_ref[...], len_b, l, cchunk)


def kernel(inputs, input_lens, W0, b0, W1, b1, W2, b2, Wa, ba, cvw, cvb,
           bnvg, bnvb, bnvm, bnvv, cnw, cnb, bnng, bnnb, bnnm, bnnv):
    B, L, D_IN = inputs.shape
    D = W0.shape[1]
    VC = cvw.shape[0]
    NC = cnw.shape[0]
    TL = 512 if L % 512 == 0 else L
    NL = L // TL
    CCHUNK = 256 if VC % 256 == 0 and NC % 256 == 0 else max(VC, NC)

    lens = input_lens.astype(jnp.int32)

    va = (bnvg / jnp.sqrt(bnvv + 1e-5))
    v_bias = (bnvb + (cvb - bnvm) * va).reshape(1, VC)
    wv = (cvw * va[:, None, None]).astype(jnp.bfloat16)
    wv0 = wv[:, :, 0].T
    wv1 = wv[:, :, 1].T
    wv2 = wv[:, :, 2].T
    na = (bnng / jnp.sqrt(bnnv + 1e-5))
    n_bias = (bnnb + (cnb - bnnm) * na).reshape(1, NC)
    wn = (cnw[:, :, 0] * na[:, None]).T.astype(jnp.bfloat16)

    full = lambda shp: pl.BlockSpec(shp, lambda b, t: (0,) * len(shp))
    e1, e2, sent, ilv, iln = pl.pallas_call(
        functools.partial(_mega_body, tl=TL, nl=NL, l=L, cchunk=CCHUNK),
        grid=(B, NL),
        in_specs=[
            pl.BlockSpec(memory_space=pltpu.SMEM),
            pl.BlockSpec(memory_space=pltpu.SMEM),
            pl.BlockSpec((1, TL, D_IN), lambda b, t: (b, t, 0)),
            full((D_IN, D)), full((1, D)),
            full((D_IN, D)), full((1, D)),
            full((D_IN, D)), full((1, D)),
            full((D, 1)),
            full((D, VC)), full((D, VC)), full((D, VC)), full((1, VC)),
            full((D, NC)), full((1, NC)),
        ],
        out_specs=[
            pl.BlockSpec((1, TL, D), lambda b, t: (b, t, 0)),
            pl.BlockSpec((1, TL, D), lambda b, t: (b, t, 0)),
            pl.BlockSpec((1, 1, D), lambda b, t: (b, 0, 0)),
            pl.BlockSpec((1, 1, VC), lambda b, t: (b, 0, 0)),
            pl.BlockSpec((1, 1, NC), lambda b, t: (b, 0, 0)),
        ],
        out_shape=[
            jax.ShapeDtypeStruct((B, L, D), jnp.float32),
            jax.ShapeDtypeStruct((B, L, D), jnp.float32),
            jax.ShapeDtypeStruct((B, 1, D), jnp.float32),
            jax.ShapeDtypeStruct((B, 1, VC), jnp.float32),
            jax.ShapeDtypeStruct((B, 1, NC), jnp.float32),
        ],
        scratch_shapes=[
            pltpu.SMEM((1, 1), jnp.float32),
            pltpu.SMEM((1, 1), jnp.float32),
            pltpu.VMEM((1, D), jnp.float32),
            pltpu.VMEM((TL, D), jnp.bfloat16),
            pltpu.VMEM((1, D), jnp.bfloat16),
            pltpu.VMEM((L, VC), jnp.float32),
            pltpu.VMEM((L, NC), jnp.float32),
        ],
        compiler_params=pltpu.CompilerParams(
            dimension_semantics=("parallel", "arbitrary")),
    )(lens, ba, inputs,
      W0.astype(jnp.bfloat16), b0.reshape(1, D),
      W1.astype(jnp.bfloat16), b1.reshape(1, D),
      W2.astype(jnp.bfloat16), b2.reshape(1, D),
      Wa, wv0, wv1, wv2, v_bias, wn, n_bias)

    return (sent.reshape(B, D), e1, e2, ilv.reshape(B, VC), iln.reshape(B, NC))
